# Initial kernel scaffold; baseline (speedup 1.0000x reference)
import jax
import jax.numpy as jnp
from jax.experimental import pallas as pl


def kernel(h_v, edge_index, h_e, W1m, b1m, W2m, b2m, W1u, b1u, W2u, b2u):
    raise NotImplementedError("write your pallas kernel here")


# speedup vs baseline: 4.2234x; 4.2234x over previous
import functools

import jax
import jax.numpy as jnp
from jax import lax
from jax.experimental import pallas as pl
from jax.experimental.pallas import tpu as pltpu
from jax.experimental.pallas import tpu_sc as plsc

N = 10000
E = 320000
H = 128
K = 128
NCHUNKS = E // K

_info = plsc.get_sparse_core_info()
NC = _info.num_cores
NS = _info.num_subcores
NW = NC * NS

TILE_STRIDE = 624
ROW_CHUNK = 128
OUT_CHUNKS = 5


def _ab_body(hv_ref, wa_ref, wb_ref, b1_ref, a_ref, b_ref):
    x = hv_ref[...]
    a_ref[...] = jnp.dot(x, wa_ref[...], preferred_element_type=jnp.float32) + b1_ref[...]
    b_ref[...] = jnp.dot(x, wb_ref[...], preferred_element_type=jnp.float32)


def _make_ab(blk=2000):
    grid = N // blk
    return pl.pallas_call(
        _ab_body,
        grid=(grid,),
        in_specs=[
            pl.BlockSpec((blk, H), lambda i: (i, 0)),
            pl.BlockSpec((H, H), lambda i: (0, 0)),
            pl.BlockSpec((H, H), lambda i: (0, 0)),
            pl.BlockSpec((1, H), lambda i: (0, 0)),
        ],
        out_specs=[
            pl.BlockSpec((blk, H), lambda i: (i, 0)),
            pl.BlockSpec((blk, H), lambda i: (i, 0)),
        ],
        out_shape=[
            jax.ShapeDtypeStruct((N, H), jnp.float32),
            jax.ShapeDtypeStruct((N, H), jnp.float32),
        ],
    )


def _c_body(he_ref, wc_ref, c_ref):
    c_ref[...] = jnp.dot(he_ref[...], wc_ref[...], preferred_element_type=jnp.float32)


def _make_c(blk=2000):
    grid = E // blk
    return pl.pallas_call(
        _c_body,
        grid=(grid,),
        in_specs=[
            pl.BlockSpec((blk, H), lambda i: (i, 0)),
            pl.BlockSpec((H, H), lambda i: (0, 0)),
        ],
        out_specs=pl.BlockSpec((blk, H), lambda i: (i, 0)),
        out_shape=jax.ShapeDtypeStruct((E, H), jnp.float32),
    )


def _sc_body(src_hbm, dst_hbm, a_hbm, b_hbm, c_hbm, out_s,
             src_v, dst_v, buf_a, buf_b, buf_c,
             s_sh, sem_a, sem_b, sem_c):
    cid = lax.axis_index("c")
    sid = lax.axis_index("s")
    wid = sid * NC + cid

    zeros16 = jnp.zeros((16,), jnp.float32)

    @pl.loop(0, K)
    def _(r):
        for k in range(H // 16):
            buf_c[r, pl.ds(k * 16, 16)] = zeros16

    r_tile = sid * TILE_STRIDE
    for j in range(OUT_CHUNKS):
        r0 = r_tile + j * ROW_CHUNK
        pltpu.sync_copy(buf_c.at[pl.ds(0, ROW_CHUNK)], s_sh.at[pl.ds(r0, ROW_CHUNK)])

    plsc.subcore_barrier()

    n_my_chunks = (NCHUNKS - wid + NW - 1) // NW

    @pl.loop(0, n_my_chunks)
    def _(i):
        base = (wid + i * NW) * K
        pltpu.sync_copy(src_hbm.at[pl.ds(base, K)], src_v)
        pltpu.sync_copy(dst_hbm.at[pl.ds(base, K)], dst_v)
        cp_a = pltpu.async_copy(a_hbm.at[src_v], buf_a, sem_a)
        cp_b = pltpu.async_copy(b_hbm.at[dst_v], buf_b, sem_b)
        cp_c = pltpu.async_copy(c_hbm.at[pl.ds(base, K)], buf_c, sem_c)
        cp_a.wait()
        cp_b.wait()
        cp_c.wait()

        @pl.loop(0, K)
        def _(r):
            for k in range(H // 16):
                sl = pl.ds(k * 16, 16)
                buf_c[r, sl] = jnp.maximum(buf_a[r, sl] + buf_b[r, sl] + buf_c[r, sl], 0.0)

        pltpu.sync_copy(buf_c, s_sh.at[dst_v], add=True)

    plsc.subcore_barrier()

    for j in range(OUT_CHUNKS):
        r0 = r_tile + j * ROW_CHUNK
        pltpu.sync_copy(s_sh.at[pl.ds(r0, ROW_CHUNK)], buf_a.at[pl.ds(0, ROW_CHUNK)])
        pltpu.sync_copy(buf_a.at[pl.ds(0, ROW_CHUNK)], out_s.at[cid, pl.ds(r0, ROW_CHUNK)])


_sc_scatter = functools.partial(
    pl.kernel,
    out_type=jax.ShapeDtypeStruct((NC, N, H), jnp.float32),
    mesh=plsc.VectorSubcoreMesh(core_axis_name="c", subcore_axis_name="s"),
    scratch_types=[
        pltpu.VMEM((K,), jnp.int32),
        pltpu.VMEM((K,), jnp.int32),
        pltpu.VMEM((K, H), jnp.float32),
        pltpu.VMEM((K, H), jnp.float32),
        pltpu.VMEM((K, H), jnp.float32),
        pltpu.VMEM_SHARED((N, H), jnp.float32),
        pltpu.SemaphoreType.DMA,
        pltpu.SemaphoreType.DMA,
        pltpu.SemaphoreType.DMA,
    ],
)(_sc_body)


def _sc_cnt_body(dst_hbm, out_cnt, dst_v, ones_b, cnt_v, stage_c, cnt_sh):
    cid = lax.axis_index("c")
    sid = lax.axis_index("s")
    wid = sid * NC + cid

    zeros16 = jnp.zeros((16,), jnp.float32)
    ones16 = jnp.full((16,), 1.0, jnp.float32)

    for k in range(K // 16):
        ones_b[pl.ds(k * 16, 16)] = ones16

    @pl.loop(0, OUT_CHUNKS * ROW_CHUNK // 16)
    def _(i):
        stage_c[pl.ds(i * 16, 16)] = zeros16

    r_tile = sid * TILE_STRIDE
    pltpu.sync_copy(stage_c, cnt_sh.at[pl.ds(r_tile, OUT_CHUNKS * ROW_CHUNK)])
    plsc.subcore_barrier()

    n_my_chunks = (NCHUNKS - wid + NW - 1) // NW

    @pl.loop(0, n_my_chunks)
    def _(i):
        base = (wid + i * NW) * K
        pltpu.sync_copy(dst_hbm.at[pl.ds(base, K)], dst_v)
        pltpu.sync_copy(ones_b, cnt_sh.at[dst_v], add=True)

    plsc.subcore_barrier()
    pltpu.sync_copy(cnt_sh.at[pl.ds(r_tile, OUT_CHUNKS * ROW_CHUNK)], cnt_v)
    pltpu.sync_copy(cnt_v, out_cnt.at[pl.ds(cid * N + r_tile, OUT_CHUNKS * ROW_CHUNK)])


_sc_count = functools.partial(
    pl.kernel,
    out_type=jax.ShapeDtypeStruct((NC * N,), jnp.float32),
    mesh=plsc.VectorSubcoreMesh(core_axis_name="c", subcore_axis_name="s"),
    scratch_types=[
        pltpu.VMEM((K,), jnp.int32),
        pltpu.VMEM((K,), jnp.float32),
        pltpu.VMEM((OUT_CHUNKS * ROW_CHUNK,), jnp.float32),
        pltpu.VMEM((OUT_CHUNKS * ROW_CHUNK,), jnp.float32),
        pltpu.VMEM_SHARED((N,), jnp.float32),
    ],
)(_sc_cnt_body)


def _final_body(sp_ref, cnt_ref, hv_ref, w2m_ref, b2m_ref, wua_ref, wub_ref,
                b1u_ref, w2u_ref, b2u_ref, out_ref):
    s = sp_ref[0] + sp_ref[1]
    c = cnt_ref[0] + cnt_ref[1]
    agg = jnp.dot(w2m_ref[...], s, preferred_element_type=jnp.float32)
    agg = (agg + c * b2m_ref[...]) / jnp.maximum(c, 1.0)
    h = jnp.dot(wua_ref[...], hv_ref[...], preferred_element_type=jnp.float32)
    h = h + jnp.dot(wub_ref[...], agg, preferred_element_type=jnp.float32) + b1u_ref[...]
    h = jnp.maximum(h, 0.0)
    out_ref[...] = jnp.dot(w2u_ref[...], h, preferred_element_type=jnp.float32) + b2u_ref[...]


def _make_final(blk=2048):
    grid = (N + blk - 1) // blk
    full = lambda i: (0, 0)
    return pl.pallas_call(
        _final_body,
        grid=(grid,),
        in_specs=[
            pl.BlockSpec((NC, H, blk), lambda i: (0, 0, i)),
            pl.BlockSpec((NC, 1, blk), lambda i: (0, 0, i)),
            pl.BlockSpec((H, blk), lambda i: (0, i)),
            pl.BlockSpec((H, H), full),
            pl.BlockSpec((H, 1), full),
            pl.BlockSpec((H, H), full),
            pl.BlockSpec((H, H), full),
            pl.BlockSpec((H, 1), full),
            pl.BlockSpec((H, H), full),
            pl.BlockSpec((H, 1), full),
        ],
        out_specs=pl.BlockSpec((H, blk), lambda i: (0, i)),
        out_shape=jax.ShapeDtypeStruct((H, N), jnp.float32),
    )


def kernel(h_v, edge_index, h_e, W1m, b1m, W2m, b2m, W1u, b1u, W2u, b2u):
    src = edge_index[0].astype(jnp.int32)
    dst = edge_index[1].astype(jnp.int32)
    wa = W1m[:H]
    wb = W1m[H:2 * H]
    wc = W1m[2 * H:]

    a_tab, b_tab = _make_ab()(h_v, wa, wb, b1m.reshape(1, H))
    c_mat = _make_c()(h_e, wc)
    s_part = _sc_scatter(src, dst, a_tab, b_tab, c_mat)
    cnt_part = _sc_count(dst)
    out_t = _make_final()(
        jnp.transpose(s_part, (0, 2, 1)), cnt_part.reshape(NC, 1, N), h_v.T,
        W2m.T, b2m.reshape(H, 1), W1u[:H].T, W1u[H:].T, b1u.reshape(H, 1),
        W2u.T, b2u.reshape(H, 1),
    )
    return out_t.T


# speedup vs baseline: 4.6550x; 1.1022x over previous
import functools

import jax
import jax.numpy as jnp
from jax import lax
from jax.experimental import pallas as pl
from jax.experimental.pallas import tpu as pltpu
from jax.experimental.pallas import tpu_sc as plsc

N = 10000
E = 320000
H = 128
K = 128
NCHUNKS = E // K

_info = plsc.get_sparse_core_info()
NC = _info.num_cores
NS = _info.num_subcores
NW = NC * NS

TILE_STRIDE = 624
ROW_CHUNK = 128
OUT_CHUNKS = 5


def _ab_body(hv_ref, wa_ref, wb_ref, b1_ref, a_ref, b_ref):
    x = hv_ref[...]
    a_ref[...] = jnp.dot(x, wa_ref[...], preferred_element_type=jnp.float32) + b1_ref[...]
    b_ref[...] = jnp.dot(x, wb_ref[...], preferred_element_type=jnp.float32)


def _make_ab(blk=2000):
    grid = N // blk
    return pl.pallas_call(
        _ab_body,
        grid=(grid,),
        in_specs=[
            pl.BlockSpec((blk, H), lambda i: (i, 0)),
            pl.BlockSpec((H, H), lambda i: (0, 0)),
            pl.BlockSpec((H, H), lambda i: (0, 0)),
            pl.BlockSpec((1, H), lambda i: (0, 0)),
        ],
        out_specs=[
            pl.BlockSpec((blk, H), lambda i: (i, 0)),
            pl.BlockSpec((blk, H), lambda i: (i, 0)),
        ],
        out_shape=[
            jax.ShapeDtypeStruct((N, H), jnp.float32),
            jax.ShapeDtypeStruct((N, H), jnp.float32),
        ],
    )


def _c_body(he_ref, wc_ref, c_ref):
    c_ref[...] = jnp.dot(he_ref[...], wc_ref[...], preferred_element_type=jnp.float32)


def _make_c(blk=2000):
    grid = E // blk
    return pl.pallas_call(
        _c_body,
        grid=(grid,),
        in_specs=[
            pl.BlockSpec((blk, H), lambda i: (i, 0)),
            pl.BlockSpec((H, H), lambda i: (0, 0)),
        ],
        out_specs=pl.BlockSpec((blk, H), lambda i: (i, 0)),
        out_shape=jax.ShapeDtypeStruct((E, H), jnp.float32),
    )


KC = 64
NCH = E // KC


def _sc_body(src_hbm, dst_hbm, a_hbm, b_hbm, c_hbm, out_s,
             src_c0, dst_c0, a0, b0, c0,
             src_c1, dst_c1, a1, b1, c1,
             s_sh, sa0, sb0, sc0, sa1, sb1, sc1, sct0, sct1):
    cid = lax.axis_index("c")
    sid = lax.axis_index("s")
    wid = sid * NC + cid
    start = (wid * NCH) // NW
    n = ((wid + 1) * NCH) // NW - start

    slots = ((src_c0, dst_c0, a0, b0, c0, sa0, sb0, sc0, sct0),
             (src_c1, dst_c1, a1, b1, c1, sa1, sb1, sc1, sct1))

    zeros16 = jnp.zeros((16,), jnp.float32)

    @pl.loop(0, KC)
    def _(r):
        for k in range(H // 16):
            c0[r, pl.ds(k * 16, 16)] = zeros16

    r_tile = sid * TILE_STRIDE
    for j in range(OUT_CHUNKS * (ROW_CHUNK // KC)):
        r0 = r_tile + j * KC
        pltpu.sync_copy(c0, s_sh.at[pl.ds(r0, KC)])

    plsc.subcore_barrier()

    def issue(j, slot):
        sv, dv, ba, bb, bc, sa, sb, sc_, sct = slots[slot]

        @pl.when(j >= 2)
        def _():
            pltpu.make_async_copy(bc, s_sh.at[dv], sct).wait()

        base = (start + j) * KC
        pltpu.sync_copy(src_hbm.at[pl.ds(base, KC)], sv)
        pltpu.sync_copy(dst_hbm.at[pl.ds(base, KC)], dv)
        pltpu.async_copy(a_hbm.at[sv], ba, sa)
        pltpu.async_copy(b_hbm.at[dv], bb, sb)
        pltpu.async_copy(c_hbm.at[pl.ds(base, KC)], bc, sc_)

    def wait_slot(slot):
        sv, dv, ba, bb, bc, sa, sb, sc_, sct = slots[slot]
        pltpu.make_async_copy(a_hbm.at[sv], ba, sa).wait()
        pltpu.make_async_copy(b_hbm.at[dv], bb, sb).wait()
        pltpu.make_async_copy(c_hbm.at[pl.ds(0, KC)], bc, sc_).wait()

    def compute_scatter(slot):
        sv, dv, ba, bb, bc, sa, sb, sc_, sct = slots[slot]

        @pl.loop(0, KC)
        def _(r):
            for k in range(H // 16):
                sl = pl.ds(k * 16, 16)
                bc[r, sl] = jnp.maximum(ba[r, sl] + bb[r, sl] + bc[r, sl], 0.0)

        pltpu.async_copy(bc, s_sh.at[dv], sct, add=True)

    issue(jnp.int32(0), 0)

    @pl.loop(0, (NCH // NW + 2) // 2)
    def _(g):
        for b in range(2):
            i = g * 2 + b

            @pl.when(i < n)
            def _():
                wait_slot(b)

                @pl.when(i + 1 < n)
                def _():
                    issue(i + 1, 1 - b)

                compute_scatter(b)

    pltpu.make_async_copy(c0, s_sh.at[dst_c0], sct0).wait()
    pltpu.make_async_copy(c1, s_sh.at[dst_c1], sct1).wait()
    plsc.subcore_barrier()

    for j in range(OUT_CHUNKS * (ROW_CHUNK // KC)):
        r0 = r_tile + j * KC
        pltpu.sync_copy(s_sh.at[pl.ds(r0, KC)], a0)
        pltpu.sync_copy(a0, out_s.at[cid, pl.ds(r0, KC)])


_sc_scatter = functools.partial(
    pl.kernel,
    out_type=jax.ShapeDtypeStruct((NC, N, H), jnp.float32),
    mesh=plsc.VectorSubcoreMesh(core_axis_name="c", subcore_axis_name="s"),
    scratch_types=[
        pltpu.VMEM((KC,), jnp.int32),
        pltpu.VMEM((KC,), jnp.int32),
        pltpu.VMEM((KC, H), jnp.float32),
        pltpu.VMEM((KC, H), jnp.float32),
        pltpu.VMEM((KC, H), jnp.float32),
        pltpu.VMEM((KC,), jnp.int32),
        pltpu.VMEM((KC,), jnp.int32),
        pltpu.VMEM((KC, H), jnp.float32),
        pltpu.VMEM((KC, H), jnp.float32),
        pltpu.VMEM((KC, H), jnp.float32),
        pltpu.VMEM_SHARED((N, H), jnp.float32),
        pltpu.SemaphoreType.DMA,
        pltpu.SemaphoreType.DMA,
        pltpu.SemaphoreType.DMA,
        pltpu.SemaphoreType.DMA,
        pltpu.SemaphoreType.DMA,
        pltpu.SemaphoreType.DMA,
        pltpu.SemaphoreType.DMA,
        pltpu.SemaphoreType.DMA,
    ],
)(_sc_body)


def _sc_cnt_body(dst_hbm, out_cnt, dst_v, ones_b, cnt_v, stage_c, cnt_sh):
    cid = lax.axis_index("c")
    sid = lax.axis_index("s")
    wid = sid * NC + cid

    zeros16 = jnp.zeros((16,), jnp.float32)
    ones16 = jnp.full((16,), 1.0, jnp.float32)

    for k in range(K // 16):
        ones_b[pl.ds(k * 16, 16)] = ones16

    @pl.loop(0, OUT_CHUNKS * ROW_CHUNK // 16)
    def _(i):
        stage_c[pl.ds(i * 16, 16)] = zeros16

    r_tile = sid * TILE_STRIDE
    pltpu.sync_copy(stage_c, cnt_sh.at[pl.ds(r_tile, OUT_CHUNKS * ROW_CHUNK)])
    plsc.subcore_barrier()

    n_my_chunks = (NCHUNKS - wid + NW - 1) // NW

    @pl.loop(0, n_my_chunks)
    def _(i):
        base = (wid + i * NW) * K
        pltpu.sync_copy(dst_hbm.at[pl.ds(base, K)], dst_v)
        pltpu.sync_copy(ones_b, cnt_sh.at[dst_v], add=True)

    plsc.subcore_barrier()
    pltpu.sync_copy(cnt_sh.at[pl.ds(r_tile, OUT_CHUNKS * ROW_CHUNK)], cnt_v)
    pltpu.sync_copy(cnt_v, out_cnt.at[pl.ds(cid * N + r_tile, OUT_CHUNKS * ROW_CHUNK)])


_sc_count = functools.partial(
    pl.kernel,
    out_type=jax.ShapeDtypeStruct((NC * N,), jnp.float32),
    mesh=plsc.VectorSubcoreMesh(core_axis_name="c", subcore_axis_name="s"),
    scratch_types=[
        pltpu.VMEM((K,), jnp.int32),
        pltpu.VMEM((K,), jnp.float32),
        pltpu.VMEM((OUT_CHUNKS * ROW_CHUNK,), jnp.float32),
        pltpu.VMEM((OUT_CHUNKS * ROW_CHUNK,), jnp.float32),
        pltpu.VMEM_SHARED((N,), jnp.float32),
    ],
)(_sc_cnt_body)


def _final_body(sp_ref, cnt_ref, hv_ref, w2m_ref, b2m_ref, wua_ref, wub_ref,
                b1u_ref, w2u_ref, b2u_ref, out_ref):
    s = sp_ref[0] + sp_ref[1]
    c = cnt_ref[0] + cnt_ref[1]
    agg = jnp.dot(w2m_ref[...], s, preferred_element_type=jnp.float32)
    agg = (agg + c * b2m_ref[...]) / jnp.maximum(c, 1.0)
    h = jnp.dot(wua_ref[...], hv_ref[...], preferred_element_type=jnp.float32)
    h = h + jnp.dot(wub_ref[...], agg, preferred_element_type=jnp.float32) + b1u_ref[...]
    h = jnp.maximum(h, 0.0)
    out_ref[...] = jnp.dot(w2u_ref[...], h, preferred_element_type=jnp.float32) + b2u_ref[...]


def _make_final(blk=2048):
    grid = (N + blk - 1) // blk
    full = lambda i: (0, 0)
    return pl.pallas_call(
        _final_body,
        grid=(grid,),
        in_specs=[
            pl.BlockSpec((NC, H, blk), lambda i: (0, 0, i)),
            pl.BlockSpec((NC, 1, blk), lambda i: (0, 0, i)),
            pl.BlockSpec((H, blk), lambda i: (0, i)),
            pl.BlockSpec((H, H), full),
            pl.BlockSpec((H, 1), full),
            pl.BlockSpec((H, H), full),
            pl.BlockSpec((H, H), full),
            pl.BlockSpec((H, 1), full),
            pl.BlockSpec((H, H), full),
            pl.BlockSpec((H, 1), full),
        ],
        out_specs=pl.BlockSpec((H, blk), lambda i: (0, i)),
        out_shape=jax.ShapeDtypeStruct((H, N), jnp.float32),
    )


def kernel(h_v, edge_index, h_e, W1m, b1m, W2m, b2m, W1u, b1u, W2u, b2u):
    src = edge_index[0].astype(jnp.int32)
    dst = edge_index[1].astype(jnp.int32)
    wa = W1m[:H]
    wb = W1m[H:2 * H]
    wc = W1m[2 * H:]

    a_tab, b_tab = _make_ab()(h_v, wa, wb, b1m.reshape(1, H))
    c_mat = _make_c()(h_e, wc)
    s_part = _sc_scatter(src, dst, a_tab, b_tab, c_mat)
    cnt_part = _sc_count(dst)
    out_t = _make_final()(
        jnp.transpose(s_part, (0, 2, 1)), cnt_part.reshape(NC, 1, N), h_v.T,
        W2m.T, b2m.reshape(H, 1), W1u[:H].T, W1u[H:].T, b1u.reshape(H, 1),
        W2u.T, b2u.reshape(H, 1),
    )
    return out_t.T


# speedup vs baseline: 5.5261x; 1.1871x over previous
import functools

import jax
import jax.numpy as jnp
from jax import lax
from jax.experimental import pallas as pl
from jax.experimental.pallas import tpu as pltpu
from jax.experimental.pallas import tpu_sc as plsc

N = 10000
E = 320000
H = 128
K = 128
NCHUNKS = E // K

_info = plsc.get_sparse_core_info()
NC = _info.num_cores
NS = _info.num_subcores
NW = NC * NS

TILE_STRIDE = 624
ROW_CHUNK = 128
OUT_CHUNKS = 5


def _ab_body(hv_ref, wa_ref, wb_ref, b1_ref, a_ref, b_ref):
    x = hv_ref[...]
    a_ref[...] = jnp.dot(x, wa_ref[...], preferred_element_type=jnp.float32) + b1_ref[...]
    b_ref[...] = jnp.dot(x, wb_ref[...], preferred_element_type=jnp.float32)


def _make_ab(blk=2000):
    grid = N // blk
    return pl.pallas_call(
        _ab_body,
        grid=(grid,),
        in_specs=[
            pl.BlockSpec((blk, H), lambda i: (i, 0)),
            pl.BlockSpec((H, H), lambda i: (0, 0)),
            pl.BlockSpec((H, H), lambda i: (0, 0)),
            pl.BlockSpec((1, H), lambda i: (0, 0)),
        ],
        out_specs=[
            pl.BlockSpec((blk, H), lambda i: (i, 0)),
            pl.BlockSpec((blk, H), lambda i: (i, 0)),
        ],
        out_shape=[
            jax.ShapeDtypeStruct((N, H), jnp.float32),
            jax.ShapeDtypeStruct((N, H), jnp.float32),
        ],
    )


def _c_body(he_ref, wc_ref, c_ref):
    c_ref[...] = jnp.dot(he_ref[...], wc_ref[...], preferred_element_type=jnp.float32)


def _make_c(blk=2000):
    grid = E // blk
    return pl.pallas_call(
        _c_body,
        grid=(grid,),
        in_specs=[
            pl.BlockSpec((blk, H), lambda i: (i, 0)),
            pl.BlockSpec((H, H), lambda i: (0, 0)),
        ],
        out_specs=pl.BlockSpec((blk, H), lambda i: (i, 0)),
        out_shape=jax.ShapeDtypeStruct((E, H), jnp.float32),
    )


KC = 64
NCH = E // KC


GRP = 8


def _sc_body(src_hbm, dst_hbm, a_hbm, b_hbm, c_hbm, out_s,
             src_g, dst_g, dst_c0, a0, b0, c0,
             dst_c1, a1, b1, c1,
             s_sh, sa0, sb0, sc0, sa1, sb1, sc1, sct0, sct1):
    cid = lax.axis_index("c")
    sid = lax.axis_index("s")
    wid = sid * NC + cid
    start = (wid * NCH) // NW
    n = ((wid + 1) * NCH) // NW - start

    slots = ((dst_c0, a0, b0, c0, sa0, sb0, sc0, sct0),
             (dst_c1, a1, b1, c1, sa1, sb1, sc1, sct1))

    zeros16 = jnp.zeros((16,), jnp.float32)

    @pl.loop(0, KC)
    def _(r):
        for k in range(H // 16):
            c0[r, pl.ds(k * 16, 16)] = zeros16

    r_tile = sid * TILE_STRIDE
    for j in range(OUT_CHUNKS * (ROW_CHUNK // KC)):
        r0 = r_tile + j * KC
        pltpu.sync_copy(c0, s_sh.at[pl.ds(r0, KC)])

    plsc.subcore_barrier()

    def issue(j, slot):
        dv, ba, bb, bc, sa, sb, sc_, sct = slots[slot]

        @pl.when(j >= 2)
        def _():
            pltpu.make_async_copy(bc, s_sh.at[dv], sct).wait()

        @pl.when(j % GRP == 0)
        def _():
            gbase = (start + j) * KC
            pltpu.sync_copy(src_hbm.at[pl.ds(gbase, GRP * KC)], src_g)
            pltpu.sync_copy(dst_hbm.at[pl.ds(gbase, GRP * KC)], dst_g)

        base = (start + j) * KC
        goff = (j % GRP) * KC
        for k in range(KC // 16):
            sl = pl.ds(goff + k * 16, 16)
            dv[pl.ds(k * 16, 16)] = dst_g[sl]
        pltpu.async_copy(a_hbm.at[src_g.at[pl.ds(goff, KC)]], ba, sa)
        pltpu.async_copy(b_hbm.at[dv], bb, sb)
        pltpu.async_copy(c_hbm.at[pl.ds(base, KC)], bc, sc_)

    def wait_slot(slot):
        dv, ba, bb, bc, sa, sb, sc_, sct = slots[slot]
        pltpu.make_async_copy(a_hbm.at[dv], ba, sa).wait()
        pltpu.make_async_copy(b_hbm.at[dv], bb, sb).wait()
        pltpu.make_async_copy(c_hbm.at[pl.ds(0, KC)], bc, sc_).wait()

    def compute_scatter(slot):
        dv, ba, bb, bc, sa, sb, sc_, sct = slots[slot]

        @pl.loop(0, KC)
        def _(r):
            for k in range(H // 16):
                sl = pl.ds(k * 16, 16)
                bc[r, sl] = jnp.maximum(ba[r, sl] + bb[r, sl] + bc[r, sl], 0.0)

        pltpu.async_copy(bc, s_sh.at[dv], sct, add=True)

    issue(jnp.int32(0), 0)

    @pl.loop(0, (NCH // NW + 2) // 2)
    def _(g):
        for b in range(2):
            i = g * 2 + b

            @pl.when(i < n)
            def _():
                wait_slot(b)

                @pl.when(i + 1 < n)
                def _():
                    issue(i + 1, 1 - b)

                compute_scatter(b)

    pltpu.make_async_copy(c0, s_sh.at[dst_c0], sct0).wait()
    pltpu.make_async_copy(c1, s_sh.at[dst_c1], sct1).wait()
    plsc.subcore_barrier()

    for j in range(OUT_CHUNKS * (ROW_CHUNK // KC)):
        r0 = r_tile + j * KC
        pltpu.sync_copy(s_sh.at[pl.ds(r0, KC)], a0)
        pltpu.sync_copy(a0, out_s.at[cid, pl.ds(r0, KC)])


_sc_scatter = functools.partial(
    pl.kernel,
    out_type=jax.ShapeDtypeStruct((NC, N, H), jnp.float32),
    mesh=plsc.VectorSubcoreMesh(core_axis_name="c", subcore_axis_name="s"),
    scratch_types=[
        pltpu.VMEM((GRP * KC,), jnp.int32),
        pltpu.VMEM((GRP * KC,), jnp.int32),
        pltpu.VMEM((KC,), jnp.int32),
        pltpu.VMEM((KC, H), jnp.float32),
        pltpu.VMEM((KC, H), jnp.float32),
        pltpu.VMEM((KC, H), jnp.float32),
        pltpu.VMEM((KC,), jnp.int32),
        pltpu.VMEM((KC, H), jnp.float32),
        pltpu.VMEM((KC, H), jnp.float32),
        pltpu.VMEM((KC, H), jnp.float32),
        pltpu.VMEM_SHARED((N, H), jnp.float32),
        pltpu.SemaphoreType.DMA,
        pltpu.SemaphoreType.DMA,
        pltpu.SemaphoreType.DMA,
        pltpu.SemaphoreType.DMA,
        pltpu.SemaphoreType.DMA,
        pltpu.SemaphoreType.DMA,
        pltpu.SemaphoreType.DMA,
        pltpu.SemaphoreType.DMA,
    ],
)(_sc_body)


def _sc_cnt_body(dst_hbm, out_cnt, dst_v, ones_b, cnt_v, stage_c, cnt_sh):
    cid = lax.axis_index("c")
    sid = lax.axis_index("s")
    wid = sid * NC + cid

    zeros16 = jnp.zeros((16,), jnp.float32)
    ones16 = jnp.full((16,), 1.0, jnp.float32)

    for k in range(K // 16):
        ones_b[pl.ds(k * 16, 16)] = ones16

    @pl.loop(0, OUT_CHUNKS * ROW_CHUNK // 16)
    def _(i):
        stage_c[pl.ds(i * 16, 16)] = zeros16

    r_tile = sid * TILE_STRIDE
    pltpu.sync_copy(stage_c, cnt_sh.at[pl.ds(r_tile, OUT_CHUNKS * ROW_CHUNK)])
    plsc.subcore_barrier()

    n_my_chunks = (NCHUNKS - wid + NW - 1) // NW

    @pl.loop(0, n_my_chunks)
    def _(i):
        base = (wid + i * NW) * K
        pltpu.sync_copy(dst_hbm.at[pl.ds(base, K)], dst_v)
        pltpu.sync_copy(ones_b, cnt_sh.at[dst_v], add=True)

    plsc.subcore_barrier()
    pltpu.sync_copy(cnt_sh.at[pl.ds(r_tile, OUT_CHUNKS * ROW_CHUNK)], cnt_v)
    pltpu.sync_copy(cnt_v, out_cnt.at[pl.ds(cid * N + r_tile, OUT_CHUNKS * ROW_CHUNK)])


_sc_count = functools.partial(
    pl.kernel,
    out_type=jax.ShapeDtypeStruct((NC * N,), jnp.float32),
    mesh=plsc.VectorSubcoreMesh(core_axis_name="c", subcore_axis_name="s"),
    scratch_types=[
        pltpu.VMEM((K,), jnp.int32),
        pltpu.VMEM((K,), jnp.float32),
        pltpu.VMEM((OUT_CHUNKS * ROW_CHUNK,), jnp.float32),
        pltpu.VMEM((OUT_CHUNKS * ROW_CHUNK,), jnp.float32),
        pltpu.VMEM_SHARED((N,), jnp.float32),
    ],
)(_sc_cnt_body)


def _final_body(sp_ref, cnt_ref, hv_ref, w2m_ref, b2m_ref, wua_ref, wub_ref,
                b1u_ref, w2u_ref, b2u_ref, out_ref):
    s = sp_ref[0] + sp_ref[1]
    c = cnt_ref[0] + cnt_ref[1]
    agg = jnp.dot(w2m_ref[...], s, preferred_element_type=jnp.float32)
    agg = (agg + c * b2m_ref[...]) / jnp.maximum(c, 1.0)
    h = jnp.dot(wua_ref[...], hv_ref[...], preferred_element_type=jnp.float32)
    h = h + jnp.dot(wub_ref[...], agg, preferred_element_type=jnp.float32) + b1u_ref[...]
    h = jnp.maximum(h, 0.0)
    out_ref[...] = jnp.dot(w2u_ref[...], h, preferred_element_type=jnp.float32) + b2u_ref[...]


def _make_final(blk=2048):
    grid = (N + blk - 1) // blk
    full = lambda i: (0, 0)
    return pl.pallas_call(
        _final_body,
        grid=(grid,),
        in_specs=[
            pl.BlockSpec((NC, H, blk), lambda i: (0, 0, i)),
            pl.BlockSpec((NC, 1, blk), lambda i: (0, 0, i)),
            pl.BlockSpec((H, blk), lambda i: (0, i)),
            pl.BlockSpec((H, H), full),
            pl.BlockSpec((H, 1), full),
            pl.BlockSpec((H, H), full),
            pl.BlockSpec((H, H), full),
            pl.BlockSpec((H, 1), full),
            pl.BlockSpec((H, H), full),
            pl.BlockSpec((H, 1), full),
        ],
        out_specs=pl.BlockSpec((H, blk), lambda i: (0, i)),
        out_shape=jax.ShapeDtypeStruct((H, N), jnp.float32),
    )


def kernel(h_v, edge_index, h_e, W1m, b1m, W2m, b2m, W1u, b1u, W2u, b2u):
    src = edge_index[0].astype(jnp.int32)
    dst = edge_index[1].astype(jnp.int32)
    wa = W1m[:H]
    wb = W1m[H:2 * H]
    wc = W1m[2 * H:]

    pad = jnp.zeros((GRP * KC,), jnp.int32)
    src_p = jnp.concatenate([src, pad])
    dst_p = jnp.concatenate([dst, pad])

    cnt_part = _sc_count(dst)
    a_tab, b_tab = _make_ab()(h_v, wa, wb, b1m.reshape(1, H))
    c_mat = _make_c()(h_e, wc)
    s_part = _sc_scatter(src_p, dst_p, a_tab, b_tab, c_mat)
    out_t = _make_final()(
        jnp.transpose(s_part, (0, 2, 1)), cnt_part.reshape(NC, 1, N), h_v.T,
        W2m.T, b2m.reshape(H, 1), W1u[:H].T, W1u[H:].T, b1u.reshape(H, 1),
        W2u.T, b2u.reshape(H, 1),
    )
    return out_t.T
